# baseline (speedup 1.0000x reference)
import functools

import jax
import jax.numpy as jnp
from jax import lax
from jax.experimental import pallas as pl
from jax.experimental.pallas import tpu as pltpu
from jax.experimental.pallas import tpu_sc as plsc

_N = 3_200_000
_T = 100_000
_PAD_T = 100_352
_ROWS = _PAD_T // 128
_SROWS = _ROWS // 16
_C = 4000
_PER_TILE = _N // 16
_CHUNKS = _PER_TILE // _C
_LOG_REG_C = 2.5e-5
_LEAF_COEF = 10.0
_CLIP = 10.0


def _sc_segment_sums(qsa_p, pb, edge_out_s, seg_s):
  mesh = plsc.VectorSubcoreMesh(core_axis_name="c", subcore_axis_name="s")

  @functools.partial(
      pl.kernel,
      out_type=jax.ShapeDtypeStruct((2, _ROWS, 128), jnp.float32),
      mesh=mesh,
      scratch_types=[
          pltpu.VMEM((_C,), jnp.float32),
          pltpu.VMEM((_C,), jnp.float32),
          pltpu.VMEM((_C,), jnp.int32),
          pltpu.VMEM((_C,), jnp.int32),
          pltpu.VMEM((_ROWS, 128), jnp.float32),
          pltpu.VMEM((16,), jnp.int32),
          pltpu.VMEM_SHARED((_ROWS, 128), jnp.float32),
          pltpu.SemaphoreType.DMA,
          pltpu.SemaphoreType.DMA,
      ],
      compiler_params=pltpu.CompilerParams(
          needs_layout_passes=False, use_tc_tiling_on_sc=False),
  )
  def k(qsa_hbm, pb_hbm, eos_hbm, seg_hbm, out_hbm,
        v0, v1, i0, i1, local, widx, acc, sem0, sem1):
    c = lax.axis_index("c")
    s = lax.axis_index("s")

    def zero_body(r, carry):
      for jj in range(8):
        local[r, pl.ds(jj * 16, 16)] = jnp.zeros((16,), jnp.float32)
      return carry

    lax.fori_loop(0, _ROWS, zero_body, 0)
    pltpu.sync_copy(local.at[pl.ds(s * _SROWS, _SROWS)],
                    acc.at[pl.ds(s * _SROWS, _SROWS)])

    def process(vals_hbm, idx_hbm):
      base = s * _PER_TILE

      def fetch(ci, vslot, islot, sem):
        off = base + ci * _C
        pltpu.async_copy(vals_hbm.at[pl.ds(off, _C)], vslot, sem)
        pltpu.async_copy(idx_hbm.at[pl.ds(off, _C)], islot, sem)

      def wait_fetch(vslot, islot, sem):
        pltpu.make_async_copy(vals_hbm.at[pl.ds(0, _C)], vslot, sem).wait()
        pltpu.make_async_copy(idx_hbm.at[pl.ds(0, _C)], islot, sem).wait()

      shift_idx = jnp.minimum(lax.iota(jnp.int32, 16) + 1, 15)
      last_mask = lax.iota(jnp.int32, 16) == 15
      gdn = lax.GatherDimensionNumbers(
          offset_dims=(), collapsed_slice_dims=(0,), start_index_map=(0,))

      def accumulate(vslot, islot):
        @plsc.parallel_loop(0, _C // 16, unroll=16)
        def _(j):
          e = jnp.exp(vslot[pl.ds(j * 16, 16)])
          cs = jnp.cumsum(e)
          d = islot[pl.ds(j * 16, 16)]
          sh = lax.gather(d, shift_idx[:, None], gdn, (1,),
                          mode=lax.GatherScatterMode.PROMISE_IN_BOUNDS)
          m = d != sh
          m_end = jnp.logical_or(m, last_mask)
          row = lax.shift_right_logical(d, 7)
          col = jnp.bitwise_and(d, 127)
          plsc.addupdate_scatter(local, [row, col], cs, mask=m_end)
          rs = lax.shift_right_logical(sh, 7)
          cl = jnp.bitwise_and(sh, 127)
          plsc.addupdate_scatter(local, [rs, cl], -cs, mask=m)

      fetch(0, v0, i0, sem0)
      fetch(1, v1, i1, sem1)

      def pair_body(k2, carry):
        wait_fetch(v0, i0, sem0)
        accumulate(v0, i0)

        @pl.when(2 * k2 + 2 < _CHUNKS)
        def _():
          fetch(2 * k2 + 2, v0, i0, sem0)

        wait_fetch(v1, i1, sem1)
        accumulate(v1, i1)

        @pl.when(2 * k2 + 3 < _CHUNKS)
        def _():
          fetch(2 * k2 + 3, v1, i1, sem1)

        return carry

      lax.fori_loop(0, _CHUNKS // 2, pair_body, 0)

      pltpu.sync_copy(idx_hbm.at[pl.ds(base, 16)], i0.at[pl.ds(0, 16)])
      pltpu.sync_copy(idx_hbm.at[pl.ds(base + _PER_TILE - 16, 16)],
                      i0.at[pl.ds(16, 16)])

    @pl.when(c == 0)
    def _():
      process(qsa_hbm, pb_hbm)

    @pl.when(c == 1)
    def _():
      process(eos_hbm, seg_hbm)

    lo = jnp.min(i0[pl.ds(0, 16)])
    hi = jnp.max(i0[pl.ds(16, 16)])

    plsc.subcore_barrier()

    def merge_body(w, carry):
      widx[pl.ds(0, 16)] = lax.iota(jnp.int32, 16) + w * 16
      pltpu.sync_copy(local.at[pl.ds(w * 16, 16)], acc.at[widx], add=True)
      return carry

    lo_w = lax.shift_right_logical(lo, 7) // 16
    hi_w = lax.shift_right_logical(hi, 7) // 16
    lax.fori_loop(lo_w, hi_w + 1, merge_body, 0)
    plsc.subcore_barrier()

    pltpu.sync_copy(acc.at[pl.ds(s * _SROWS, _SROWS)],
                    local.at[pl.ds(0, _SROWS)])

    @pl.when(c == 0)
    def _():
      pltpu.sync_copy(local.at[pl.ds(0, _SROWS)],
                      out_hbm.at[0, pl.ds(s * _SROWS, _SROWS)])

    @pl.when(c == 1)
    def _():
      pltpu.sync_copy(local.at[pl.ds(0, _SROWS)],
                      out_hbm.at[1, pl.ds(s * _SROWS, _SROWS)])

  return k(qsa_p, pb, edge_out_s, seg_s)


def _tc_loss_body(acc_ref, rw_ref, dn_ref, out_ref):
  ei = acc_ref[0]
  eo = acc_ref[1]
  rwv = rw_ref[...]
  dnv = dn_ref[...]
  done_b = (dnv > 0.5).astype(jnp.float32)
  inflow = jnp.log(ei + _LOG_REG_C)
  opr = jnp.log(_LOG_REG_C + rwv + eo * (1.0 - done_b))
  l = (inflow - opr) ** 2
  l = jnp.minimum(l, _CLIP)
  rows, cols = rw_ref.shape
  li = (lax.broadcasted_iota(jnp.int32, (rows, cols), 0) * cols
        + lax.broadcasted_iota(jnp.int32, (rows, cols), 1))
  valid = (li < _T).astype(jnp.float32)
  term_num = jnp.sum(l * done_b)
  term_den = jnp.sum(done_b)
  flow_num = jnp.sum(l * (1.0 - done_b) * valid)
  flow_den = jnp.sum((1.0 - done_b) * valid)
  out_ref[0, 0] = (term_num / (term_den + 1e-20) * _LEAF_COEF
                   + flow_num / (flow_den + 1e-20))


def _tc_loss(acc, reward, done, interpret=False):
  acc3 = acc.reshape(2, _PAD_T // 128, 128)
  pad = _PAD_T - _T
  rw = jnp.pad(reward, (0, pad)).reshape(_PAD_T // 128, 128)
  dn = jnp.pad(done, (0, pad)).reshape(_PAD_T // 128, 128)
  out = pl.pallas_call(
      _tc_loss_body,
      out_shape=jax.ShapeDtypeStruct((1, 1), jnp.float32),
      out_specs=pl.BlockSpec(memory_space=pltpu.MemorySpace.SMEM),
      interpret=interpret,
  )(acc3, rw, dn)
  return out.reshape(())


def kernel(qsa_p, pb, edge_out_s, seg_s, reward, done):
  acc = _sc_segment_sums(qsa_p, pb.astype(jnp.int32),
                         edge_out_s, seg_s.astype(jnp.int32))
  return _tc_loss(acc, reward, done)


# speedup vs baseline: 1.0595x; 1.0595x over previous
import functools

import jax
import jax.numpy as jnp
from jax import lax
from jax.experimental import pallas as pl
from jax.experimental.pallas import tpu as pltpu
from jax.experimental.pallas import tpu_sc as plsc

_N = 3_200_000
_T = 100_000
_PAD_T = 100_352
_ROWS = _PAD_T // 128
_SROWS = _ROWS // 16
_C = 4000
_PER_TILE = _N // 16
_CHUNKS = _PER_TILE // _C
_LOG_REG_C = 2.5e-5
_LEAF_COEF = 10.0
_CLIP = 10.0


def _sc_segment_sums(qsa_p, pb, edge_out_s, seg_s):
  mesh = plsc.VectorSubcoreMesh(core_axis_name="c", subcore_axis_name="s")

  @functools.partial(
      pl.kernel,
      out_type=jax.ShapeDtypeStruct((2, _ROWS, 128), jnp.float32),
      mesh=mesh,
      scratch_types=[
          pltpu.VMEM((_C,), jnp.float32),
          pltpu.VMEM((_C,), jnp.float32),
          pltpu.VMEM((_C + 16,), jnp.int32),
          pltpu.VMEM((_C + 16,), jnp.int32),
          pltpu.VMEM((_ROWS, 128), jnp.float32),
          pltpu.VMEM((32,), jnp.int32),
          pltpu.VMEM((16,), jnp.int32),
          pltpu.VMEM_SHARED((_ROWS, 128), jnp.float32),
          pltpu.SemaphoreType.DMA,
          pltpu.SemaphoreType.DMA,
      ],
      compiler_params=pltpu.CompilerParams(
          needs_layout_passes=False, use_tc_tiling_on_sc=False),
  )
  def k(qsa_hbm, pb_hbm, eos_hbm, seg_hbm, out_hbm,
        v0, v1, i0, i1, local, bnd, widx, acc, sem0, sem1):
    c = lax.axis_index("c")
    s = lax.axis_index("s")
    base = s * _PER_TILE

    def fetch(vals_hbm, idx_hbm, ci, vslot, islot, sem):
      off = base + ci * _C
      pltpu.async_copy(vals_hbm.at[pl.ds(off, _C)], vslot, sem)
      pltpu.async_copy(idx_hbm.at[pl.ds(off, _C)], islot.at[pl.ds(0, _C)],
                       sem)

    def prologue(vals_hbm, idx_hbm):
      pltpu.sync_copy(idx_hbm.at[pl.ds(base, 16)], bnd.at[pl.ds(0, 16)])
      pltpu.sync_copy(idx_hbm.at[pl.ds(base + _PER_TILE - 16, 16)],
                      bnd.at[pl.ds(16, 16)])
      fetch(vals_hbm, idx_hbm, 0, v0, i0, sem0)
      fetch(vals_hbm, idx_hbm, 1, v1, i1, sem1)

    @pl.when(c == 0)
    def _():
      prologue(qsa_hbm, pb_hbm)

    @pl.when(c == 1)
    def _():
      prologue(eos_hbm, seg_hbm)

    lo = jnp.min(bnd[pl.ds(0, 16)])
    hi = jnp.max(bnd[pl.ds(16, 16)])
    lo_r = lax.shift_right_logical(lo, 7)
    hi_r = lax.shift_right_logical(hi, 7)
    lo_w = lo_r // 16
    hi_w = hi_r // 16

    def zero_body(r, carry):
      for jj in range(8):
        local[r, pl.ds(jj * 16, 16)] = jnp.zeros((16,), jnp.float32)
      return carry

    lax.fori_loop(0, _SROWS, zero_body, 0)
    pltpu.sync_copy(local.at[pl.ds(0, _SROWS)],
                    acc.at[pl.ds(s * _SROWS, _SROWS)])
    lax.fori_loop(lo_w * 16, (hi_w + 1) * 16, zero_body, 0)

    def main_loop(vals_hbm, idx_hbm):
      def wait_fetch(vslot, islot, sem):
        pltpu.make_async_copy(vals_hbm.at[pl.ds(0, _C)], vslot, sem).wait()
        pltpu.make_async_copy(idx_hbm.at[pl.ds(0, _C)],
                              islot.at[pl.ds(0, _C)], sem).wait()

      last_mask = lax.iota(jnp.int32, 16) == 15
      not_last = lax.iota(jnp.int32, 16) < 15

      def accumulate(vslot, islot):
        @plsc.parallel_loop(0, _C // 16, unroll=8)
        def _(j):
          e = jnp.exp(vslot[pl.ds(j * 16, 16)])
          cs = jnp.cumsum(e)
          d = islot[pl.ds(j * 16, 16)]
          sh = islot[pl.ds(j * 16 + 1, 16)]
          m = d != sh
          m_end = jnp.logical_or(m, last_mask)
          m_sub = jnp.logical_and(m, not_last)
          row = lax.shift_right_logical(d, 7)
          col = jnp.bitwise_and(d, 127)
          plsc.addupdate_scatter(local, [row, col], cs, mask=m_end)
          rs = lax.shift_right_logical(sh, 7)
          cl = jnp.bitwise_and(sh, 127)
          plsc.addupdate_scatter(local, [rs, cl], -cs, mask=m_sub)

      def pair_body(k2, carry):
        wait_fetch(v0, i0, sem0)
        accumulate(v0, i0)

        @pl.when(2 * k2 + 2 < _CHUNKS)
        def _():
          fetch(vals_hbm, idx_hbm, 2 * k2 + 2, v0, i0, sem0)

        wait_fetch(v1, i1, sem1)
        accumulate(v1, i1)

        @pl.when(2 * k2 + 3 < _CHUNKS)
        def _():
          fetch(vals_hbm, idx_hbm, 2 * k2 + 3, v1, i1, sem1)

        return carry

      lax.fori_loop(0, _CHUNKS // 2, pair_body, 0)

    @pl.when(c == 0)
    def _():
      main_loop(qsa_hbm, pb_hbm)

    @pl.when(c == 1)
    def _():
      main_loop(eos_hbm, seg_hbm)

    plsc.subcore_barrier()

    def merge_body(w, carry):
      widx[pl.ds(0, 16)] = lax.iota(jnp.int32, 16) + w * 16
      pltpu.sync_copy(local.at[pl.ds(w * 16, 16)], acc.at[widx.at[pl.ds(0, 16)]],
                      add=True)
      return carry

    lo_w = lo_r // 16
    hi_w = hi_r // 16
    lax.fori_loop(lo_w, hi_w + 1, merge_body, 0)
    plsc.subcore_barrier()

    pltpu.sync_copy(acc.at[pl.ds(s * _SROWS, _SROWS)],
                    local.at[pl.ds(0, _SROWS)])

    @pl.when(c == 0)
    def _():
      pltpu.sync_copy(local.at[pl.ds(0, _SROWS)],
                      out_hbm.at[0, pl.ds(s * _SROWS, _SROWS)])

    @pl.when(c == 1)
    def _():
      pltpu.sync_copy(local.at[pl.ds(0, _SROWS)],
                      out_hbm.at[1, pl.ds(s * _SROWS, _SROWS)])

  return k(qsa_p, pb, edge_out_s, seg_s)


def _tc_loss_body(acc_ref, rw_ref, dn_ref, out_ref):
  ei = acc_ref[0]
  eo = acc_ref[1]
  rwv = rw_ref[...]
  dnv = dn_ref[...]
  done_b = (dnv > 0.5).astype(jnp.float32)
  inflow = jnp.log(ei + _LOG_REG_C)
  opr = jnp.log(_LOG_REG_C + rwv + eo * (1.0 - done_b))
  l = (inflow - opr) ** 2
  l = jnp.minimum(l, _CLIP)
  rows, cols = rw_ref.shape
  li = (lax.broadcasted_iota(jnp.int32, (rows, cols), 0) * cols
        + lax.broadcasted_iota(jnp.int32, (rows, cols), 1))
  valid = (li < _T).astype(jnp.float32)
  term_num = jnp.sum(l * done_b)
  term_den = jnp.sum(done_b)
  flow_num = jnp.sum(l * (1.0 - done_b) * valid)
  flow_den = jnp.sum((1.0 - done_b) * valid)
  out_ref[0, 0] = (term_num / (term_den + 1e-20) * _LEAF_COEF
                   + flow_num / (flow_den + 1e-20))


def _tc_loss(acc, reward, done, interpret=False):
  acc3 = acc.reshape(2, _PAD_T // 128, 128)
  pad = _PAD_T - _T
  rw = jnp.pad(reward, (0, pad)).reshape(_PAD_T // 128, 128)
  dn = jnp.pad(done, (0, pad)).reshape(_PAD_T // 128, 128)
  out = pl.pallas_call(
      _tc_loss_body,
      out_shape=jax.ShapeDtypeStruct((1, 1), jnp.float32),
      out_specs=pl.BlockSpec(memory_space=pltpu.MemorySpace.SMEM),
      interpret=interpret,
  )(acc3, rw, dn)
  return out.reshape(())


def kernel(qsa_p, pb, edge_out_s, seg_s, reward, done):
  acc = _sc_segment_sums(qsa_p, pb.astype(jnp.int32),
                         edge_out_s, seg_s.astype(jnp.int32))
  return _tc_loss(acc, reward, done)
